# baseline (speedup 1.0000x reference)
import jax
import jax.numpy as jnp
from jax import lax
from jax.experimental import pallas as pl
from jax.experimental.pallas import tpu as pltpu
from jax.experimental.pallas import tpu_sc as plsc

N = 10000
E = 160000
H = 256
HW = 128
CN = 16
NC = 64
EP = 163840
BLK = 128
NBLK = EP // 16 // BLK
HB = NBLK // 2
ACC_ROWS = 10112
SLAB = ACC_ROWS // 16
WB_LAST = N - 15 * SLAB

_sc_mesh = plsc.VectorSubcoreMesh(core_axis_name="c", subcore_axis_name="s")


def _sc_body(h_hbm, src_hbm, dst_hbm, zeros_hbm, ones_hbm, sum_hbm, cnt_hbm,
             acc_sh, srcb, dstb, rows0, rows1,
             gsem0, gsem1, ssem0, ssem1):
    cid = lax.axis_index("c")
    sid = lax.axis_index("s")
    slab = sid * SLAB
    wchunk = sid * NBLK
    pltpu.sync_copy(dst_hbm.at[pl.ds(sid * NBLK, NBLK)], dstb)
    pltpu.sync_copy(zeros_hbm, acc_sh.at[pl.ds(slab, SLAB)])
    plsc.subcore_barrier()

    def _phase1(h_half):
        for half in range(2):
            pltpu.sync_copy(src_hbm.at[pl.ds(wchunk + half * HB, HB)], srcb)
            base = half * HB
            pltpu.async_copy(h_half.at[srcb.at[0]], rows0, gsem0)

            @pl.loop(0, HB, step=2)
            def _(b):
                pltpu.async_copy(h_half.at[srcb.at[b + 1]], rows1, gsem1)
                pltpu.make_async_copy(h_half.at[srcb.at[b]], rows0, gsem0).wait()
                s0 = pltpu.async_copy(rows0, acc_sh.at[dstb.at[base + b]],
                                      ssem0, add=True)
                pltpu.make_async_copy(h_half.at[srcb.at[b + 1]], rows1,
                                      gsem1).wait()
                s1 = pltpu.async_copy(rows1, acc_sh.at[dstb.at[base + b + 1]],
                                      ssem1, add=True)
                s0.wait()

                @pl.when(b + 2 < HB)
                def _():
                    pltpu.async_copy(h_half.at[srcb.at[b + 2]], rows0, gsem0)

                s1.wait()

    @pl.when(cid == 0)
    def _():
        _phase1(h_hbm.at[:, pl.ds(0, HW)])

    @pl.when(cid == 1)
    def _():
        _phase1(h_hbm.at[:, pl.ds(HW, HW)])

    plsc.subcore_barrier()

    @pl.when(sid < 15)
    def _():
        pltpu.sync_copy(acc_sh.at[pl.ds(slab, SLAB)],
                        sum_hbm.at[pl.ds(cid * N + slab, SLAB)])

    @pl.when(sid == 15)
    def _():
        pltpu.sync_copy(acc_sh.at[pl.ds(15 * SLAB, WB_LAST)],
                        sum_hbm.at[pl.ds(cid * N + 15 * SLAB, WB_LAST)])

    pltpu.sync_copy(zeros_hbm, acc_sh.at[pl.ds(slab, SLAB)])
    pltpu.sync_copy(ones_hbm, rows0)
    plsc.subcore_barrier()

    @pl.loop(0, HB)
    def _(b):
        pltpu.sync_copy(rows0, acc_sh.at[dstb.at[cid * HB + b]], add=True)

    plsc.subcore_barrier()

    @pl.when(sid < 15)
    def _():
        pltpu.sync_copy(acc_sh.at[pl.ds(slab, SLAB)],
                        cnt_hbm.at[pl.ds(cid * N + slab, SLAB)])

    @pl.when(sid == 15)
    def _():
        pltpu.sync_copy(acc_sh.at[pl.ds(15 * SLAB, WB_LAST)],
                        cnt_hbm.at[pl.ds(cid * N + 15 * SLAB, WB_LAST)])


@jax.jit
def _sc_aggregate(h, src_p, dstm, zeros, ones):
    k = pl.kernel(
        _sc_body,
        out_type=(jax.ShapeDtypeStruct((2 * N, HW), jnp.float32),
                  jax.ShapeDtypeStruct((2 * N, HW), jnp.float32)),
        mesh=_sc_mesh,
        scratch_types=[
            pltpu.VMEM_SHARED((ACC_ROWS, HW), jnp.float32),
            pltpu.VMEM((HB, BLK), jnp.int32),
            pltpu.VMEM((NBLK, BLK), jnp.int32),
            pltpu.VMEM((BLK, HW), jnp.float32),
            pltpu.VMEM((BLK, HW), jnp.float32),
            pltpu.SemaphoreType.DMA,
            pltpu.SemaphoreType.DMA,
            pltpu.SemaphoreType.DMA,
            pltpu.SemaphoreType.DMA,
        ],
    )
    return k(h, src_p, dstm, zeros, ones)


RB = 1000


def _tc_body(h_ref, a_ref, b_ref, c0_ref, c1_ref, sw_ref, wf_ref, o_ref):
    denom = c0_ref[:, 0:1] + c1_ref[:, 0:1] + 1.0
    summed = jnp.concatenate([a_ref[...], b_ref[...]], axis=1)
    hm = (h_ref[...] + summed) / denom
    scores = lax.dot_general(hm, sw_ref[...], (((1,), (1,)), ((), ())))
    m = jnp.max(scores, axis=1, keepdims=True)
    iota = lax.broadcasted_iota(jnp.int32, scores.shape, 1)
    idx = jnp.min(jnp.where(scores == m, iota, 2**30), axis=1, keepdims=True)
    logits = lax.dot_general(hm.astype(jnp.bfloat16),
                             wf_ref[...].astype(jnp.bfloat16),
                             (((1,), (1,)), ((), ())),
                             preferred_element_type=jnp.float32)
    acc = jnp.zeros((RB, NC), jnp.float32)
    for e in range(CN):
        acc = acc + jnp.where(idx == e, logits[:, e * NC:(e + 1) * NC], 0.0)
    o_ref[...] = acc


@jax.jit
def _tc_experts(h, sums, cnt, structure_W, task_Wf):
    return pl.pallas_call(
        _tc_body,
        grid=(N // RB,),
        in_specs=[
            pl.BlockSpec((RB, H), lambda i: (i, 0)),
            pl.BlockSpec((RB, HW), lambda i: (i, 0)),
            pl.BlockSpec((RB, HW), lambda i: (i + N // RB, 0)),
            pl.BlockSpec((RB, HW), lambda i: (i, 0)),
            pl.BlockSpec((RB, HW), lambda i: (i + N // RB, 0)),
            pl.BlockSpec((CN, H), lambda i: (0, 0)),
            pl.BlockSpec((CN * NC, H), lambda i: (0, 0)),
        ],
        out_specs=pl.BlockSpec((RB, NC), lambda i: (i, 0)),
        out_shape=jax.ShapeDtypeStruct((N, NC), jnp.float32),
    )(h, sums, sums, cnt, cnt, structure_W, task_Wf)


def kernel(h, edge_index, structure_W, task_W):
    src = edge_index[0].astype(jnp.int32)
    dst = edge_index[1].astype(jnp.int32)
    pad = EP - E
    src_p = jnp.concatenate([src, jnp.zeros((pad,), jnp.int32)]).reshape(
        EP // BLK, BLK)
    dstm = jnp.concatenate([dst, jnp.full((pad,), N, jnp.int32)]).reshape(
        EP // BLK, BLK)
    zeros = jnp.zeros((SLAB, HW), jnp.float32)
    ones = jnp.ones((BLK, HW), jnp.float32)
    sums, cnt = _sc_aggregate(h, src_p, dstm, zeros, ones)
    return _tc_experts(h, sums, cnt, structure_W, task_W.reshape(CN * NC, H))


# speedup vs baseline: 1.0325x; 1.0325x over previous
import jax
import jax.numpy as jnp
from jax import lax
from jax.experimental import pallas as pl
from jax.experimental.pallas import tpu as pltpu
from jax.experimental.pallas import tpu_sc as plsc

N = 10000
E = 160000
H = 256
HW = 128
CN = 16
NC = 64
EP = 163840
BLK = 128
NBLK = EP // 16 // BLK
HB = NBLK // 2
ACC_ROWS = 10112
SLAB = ACC_ROWS // 16
WB_LAST = N - 15 * SLAB

_sc_mesh = plsc.VectorSubcoreMesh(core_axis_name="c", subcore_axis_name="s")


def _sc_body(h_hbm, src_hbm, dst_hbm, zeros_hbm, ones_hbm, sum_hbm, cnt_hbm,
             acc_sh, srcb, dstb, rows0, rows1,
             gsem0, gsem1):
    cid = lax.axis_index("c")
    sid = lax.axis_index("s")
    slab = sid * SLAB
    wchunk = sid * NBLK
    pltpu.sync_copy(dst_hbm.at[pl.ds(sid * NBLK, NBLK)], dstb)
    pltpu.sync_copy(zeros_hbm, acc_sh.at[pl.ds(slab, SLAB)])
    plsc.subcore_barrier()

    def _phase1(h_half):
        for half in range(2):
            pltpu.sync_copy(src_hbm.at[pl.ds(wchunk + half * HB, HB)], srcb)
            base = half * HB
            pltpu.async_copy(h_half.at[srcb.at[0]], rows0, gsem0)

            @pl.loop(0, HB, step=2)
            def _(b):
                pltpu.async_copy(h_half.at[srcb.at[b + 1]], rows1, gsem1)
                pltpu.make_async_copy(h_half.at[srcb.at[b]], rows0, gsem0).wait()
                pltpu.sync_copy(rows0, acc_sh.at[dstb.at[base + b]], add=True)

                @pl.when(b + 2 < HB)
                def _():
                    pltpu.async_copy(h_half.at[srcb.at[b + 2]], rows0, gsem0)

                pltpu.make_async_copy(h_half.at[srcb.at[b + 1]], rows1,
                                      gsem1).wait()
                pltpu.sync_copy(rows1, acc_sh.at[dstb.at[base + b + 1]],
                                add=True)

    @pl.when(cid == 0)
    def _():
        _phase1(h_hbm.at[:, pl.ds(0, HW)])

    @pl.when(cid == 1)
    def _():
        _phase1(h_hbm.at[:, pl.ds(HW, HW)])

    plsc.subcore_barrier()

    @pl.when(sid < 15)
    def _():
        pltpu.sync_copy(acc_sh.at[pl.ds(slab, SLAB)],
                        sum_hbm.at[pl.ds(cid * N + slab, SLAB)])

    @pl.when(sid == 15)
    def _():
        pltpu.sync_copy(acc_sh.at[pl.ds(15 * SLAB, WB_LAST)],
                        sum_hbm.at[pl.ds(cid * N + 15 * SLAB, WB_LAST)])

    pltpu.sync_copy(zeros_hbm, acc_sh.at[pl.ds(slab, SLAB)])
    pltpu.sync_copy(ones_hbm, rows0)
    plsc.subcore_barrier()

    @pl.loop(0, HB)
    def _(b):
        pltpu.sync_copy(rows0, acc_sh.at[dstb.at[cid * HB + b]], add=True)

    plsc.subcore_barrier()

    @pl.when(sid < 15)
    def _():
        pltpu.sync_copy(acc_sh.at[pl.ds(slab, SLAB)],
                        cnt_hbm.at[pl.ds(cid * N + slab, SLAB)])

    @pl.when(sid == 15)
    def _():
        pltpu.sync_copy(acc_sh.at[pl.ds(15 * SLAB, WB_LAST)],
                        cnt_hbm.at[pl.ds(cid * N + 15 * SLAB, WB_LAST)])


@jax.jit
def _sc_aggregate(h, src_p, dstm, zeros, ones):
    k = pl.kernel(
        _sc_body,
        out_type=(jax.ShapeDtypeStruct((2 * N, HW), jnp.float32),
                  jax.ShapeDtypeStruct((2 * N, HW), jnp.float32)),
        mesh=_sc_mesh,
        scratch_types=[
            pltpu.VMEM_SHARED((ACC_ROWS, HW), jnp.float32),
            pltpu.VMEM((HB, BLK), jnp.int32),
            pltpu.VMEM((NBLK, BLK), jnp.int32),
            pltpu.VMEM((BLK, HW), jnp.float32),
            pltpu.VMEM((BLK, HW), jnp.float32),
            pltpu.SemaphoreType.DMA,
            pltpu.SemaphoreType.DMA,
        ],
    )
    return k(h, src_p, dstm, zeros, ones)


RB = 1000


def _tc_body(h_ref, a_ref, b_ref, c0_ref, c1_ref, sw_ref, wf_ref, o_ref):
    denom = c0_ref[:, 0:1] + c1_ref[:, 0:1] + 1.0
    summed = jnp.concatenate([a_ref[...], b_ref[...]], axis=1)
    hm = (h_ref[...] + summed) / denom
    scores = lax.dot_general(hm, sw_ref[...], (((1,), (1,)), ((), ())))
    m = jnp.max(scores, axis=1, keepdims=True)
    iota = lax.broadcasted_iota(jnp.int32, scores.shape, 1)
    idx = jnp.min(jnp.where(scores == m, iota, 2**30), axis=1, keepdims=True)
    logits = lax.dot_general(hm.astype(jnp.bfloat16),
                             wf_ref[...].astype(jnp.bfloat16),
                             (((1,), (1,)), ((), ())),
                             preferred_element_type=jnp.float32)
    acc = jnp.zeros((RB, NC), jnp.float32)
    for e in range(CN):
        acc = acc + jnp.where(idx == e, logits[:, e * NC:(e + 1) * NC], 0.0)
    o_ref[...] = acc


@jax.jit
def _tc_experts(h, sums, cnt, structure_W, task_Wf):
    return pl.pallas_call(
        _tc_body,
        grid=(N // RB,),
        in_specs=[
            pl.BlockSpec((RB, H), lambda i: (i, 0)),
            pl.BlockSpec((RB, HW), lambda i: (i, 0)),
            pl.BlockSpec((RB, HW), lambda i: (i + N // RB, 0)),
            pl.BlockSpec((RB, HW), lambda i: (i, 0)),
            pl.BlockSpec((RB, HW), lambda i: (i + N // RB, 0)),
            pl.BlockSpec((CN, H), lambda i: (0, 0)),
            pl.BlockSpec((CN * NC, H), lambda i: (0, 0)),
        ],
        out_specs=pl.BlockSpec((RB, NC), lambda i: (i, 0)),
        out_shape=jax.ShapeDtypeStruct((N, NC), jnp.float32),
    )(h, sums, sums, cnt, cnt, structure_W, task_Wf)


def kernel(h, edge_index, structure_W, task_W):
    src = edge_index[0].astype(jnp.int32)
    dst = edge_index[1].astype(jnp.int32)
    pad = EP - E
    src_p = jnp.concatenate([src, jnp.zeros((pad,), jnp.int32)]).reshape(
        EP // BLK, BLK)
    dstm = jnp.concatenate([dst, jnp.full((pad,), N, jnp.int32)]).reshape(
        EP // BLK, BLK)
    zeros = jnp.zeros((SLAB, HW), jnp.float32)
    ones = jnp.ones((BLK, HW), jnp.float32)
    sums, cnt = _sc_aggregate(h, src_p, dstm, zeros, ones)
    return _tc_experts(h, sums, cnt, structure_W, task_W.reshape(CN * NC, H))
